# baseline (speedup 1.0000x reference)
import jax
import jax.numpy as jnp
from jax import lax
from jax.experimental import pallas as pl
from jax.experimental.pallas import tpu as pltpu
from jax.experimental.pallas import tpu_sc as plsc

N = 10000
E = 320000
D = 128
H = 64

NC = 2
NS = 16
NW = NC * NS
EPW = E // NW
CH = 500
NCHUNK = EPW // CH
CH0 = 250
NCH0 = (E // NS) // CH0
NP = 10240
RPT = NP // NS

K = 8

BN = 2048
GRID = NP // BN


def _gather(hn_hbm, srcv, rows, gsem, j, k):
    return pltpu.make_async_copy(hn_hbm.at[srcv.at[j]], rows.at[k], gsem[k])


def _scat(rows, acc, dstv, ssem, j, k):
    return pltpu.make_async_copy(rows.at[k], acc.at[dstv.at[j]], ssem[k])


def _sc_body(K, hn_hbm, src_hbm, dst_hbm, z_hbm,
             parts_hbm,
             srcv, dstv, rows, acc, *sems):
    NBLK = NCHUNK // K
    gsem, ssem = sems[:K], sems[K:2 * K]
    c = lax.axis_index("c")
    s = lax.axis_index("s")
    wid = c * NS + s
    pltpu.sync_copy(src_hbm.at[wid], srcv)
    pltpu.sync_copy(dst_hbm.at[wid], dstv)
    pltpu.sync_copy(z_hbm, acc.at[pl.ds(s * RPT, RPT)])
    for k in range(K):
        _gather(hn_hbm, srcv, rows, gsem, k, k).start()
    plsc.subcore_barrier()

    def block(jb, carry, issue_next):
        base = jb * K
        for k in range(K):
            j = base + k
            _gather(hn_hbm, srcv, rows, gsem, j, k).wait()
            _scat(rows, acc, dstv, ssem, j, k).start(add=True)
        if issue_next:
            for k in range(K):
                j = base + k
                _scat(rows, acc, dstv, ssem, j, k).wait()
                _gather(hn_hbm, srcv, rows, gsem, j + K, k).start()
        return carry

    lax.fori_loop(0, NBLK - 1, lambda jb, cy: block(jb, cy, True), 0)
    block(NBLK - 1, 0, False)
    for k in range(K):
        _scat(rows, acc, dstv, ssem, (NBLK - 1) * K + k, k).wait()
    plsc.subcore_barrier()
    pltpu.sync_copy(acc.at[pl.ds(s * RPT, RPT)],
                    parts_hbm.at[c, pl.ds(s * RPT, RPT)])


def _sc0_body(K, xlo_hbm, xhi_hbm, src_hbm, dst_hbm, z_hbm, z1_hbm,
              ones_hbm,
              parts_hbm, deg_hbm,
              srcv, dstv, rows, onesv, acc, dacc, *sems):
    NBLK = NCH0 // K
    gsem, ssem, dsem = sems[:K], sems[K:2 * K], sems[2 * K:3 * K]
    c = lax.axis_index("c")
    s = lax.axis_index("s")
    pltpu.sync_copy(src_hbm.at[s], srcv)
    pltpu.sync_copy(dst_hbm.at[s], dstv)
    pltpu.sync_copy(ones_hbm, onesv)
    pltpu.sync_copy(z_hbm, acc.at[pl.ds(s * RPT, RPT)])
    pltpu.sync_copy(z1_hbm, dacc.at[pl.ds(s * RPT, RPT)])

    def dscat(j, k):
        return pltpu.make_async_copy(onesv, dacc.at[dstv.at[j]], dsem[k])

    def pipe(xc, with_deg):
        for k in range(K):
            _gather(xc, srcv, rows, gsem, k, k).start()
        plsc.subcore_barrier()

        def block(jb, carry, issue_next):
            base = jb * K
            for k in range(K):
                j = base + k
                _gather(xc, srcv, rows, gsem, j, k).wait()
                _scat(rows, acc, dstv, ssem, j, k).start(add=True)
                if with_deg:
                    dscat(j, k).start(add=True)
            if issue_next:
                for k in range(K):
                    j = base + k
                    _scat(rows, acc, dstv, ssem, j, k).wait()
                    if with_deg:
                        dscat(j, k).wait()
                    _gather(xc, srcv, rows, gsem, j + K, k).start()
            return carry

        lax.fori_loop(0, NBLK - 1, lambda jb, cy: block(jb, cy, True), 0)
        block(NBLK - 1, 0, False)
        for k in range(K):
            j = (NBLK - 1) * K + k
            _scat(rows, acc, dstv, ssem, j, k).wait()
            if with_deg:
                dscat(j, k).wait()
        plsc.subcore_barrier()

    @pl.when(c == 0)
    def _():
        pipe(xlo_hbm, True)

    @pl.when(c == 1)
    def _():
        pipe(xhi_hbm, False)
    pltpu.sync_copy(acc.at[pl.ds(s * RPT, RPT)],
                    parts_hbm.at[c, pl.ds(s * RPT, RPT)])

    @pl.when(c == 0)
    def _():
        pltpu.sync_copy(dacc.at[pl.ds(s * RPT, RPT)],
                        deg_hbm.at[pl.ds(s * RPT, RPT)])


def _sc_body_deg(K, hn_hbm, src_hbm, dst_hbm, z_hbm, z1_hbm, ones_hbm,
                 parts_hbm, degp_hbm,
                 srcv, dstv, rows, onesv, acc, dacc, *sems):
    NBLK = NCHUNK // K
    gsem, ssem, dsem = sems[:K], sems[K:2 * K], sems[2 * K:3 * K]
    c = lax.axis_index("c")
    s = lax.axis_index("s")
    wid = c * NS + s
    pltpu.sync_copy(src_hbm.at[wid], srcv)
    pltpu.sync_copy(dst_hbm.at[wid], dstv)
    pltpu.sync_copy(ones_hbm, onesv)
    pltpu.sync_copy(z_hbm, acc.at[pl.ds(s * RPT, RPT)])
    pltpu.sync_copy(z1_hbm, dacc.at[pl.ds(s * RPT, RPT)])
    for k in range(K):
        _gather(hn_hbm, srcv, rows, gsem, k, k).start()
    plsc.subcore_barrier()

    def dscat(j, k):
        return pltpu.make_async_copy(onesv, dacc.at[dstv.at[j]], dsem[k])

    def block(jb, carry, issue_next):
        base = jb * K
        for k in range(K):
            j = base + k
            _gather(hn_hbm, srcv, rows, gsem, j, k).wait()
            _scat(rows, acc, dstv, ssem, j, k).start(add=True)
            dscat(j, k).start(add=True)
        if issue_next:
            for k in range(K):
                j = base + k
                _scat(rows, acc, dstv, ssem, j, k).wait()
                dscat(j, k).wait()
                _gather(hn_hbm, srcv, rows, gsem, j + K, k).start()
        return carry

    lax.fori_loop(0, NBLK - 1, lambda jb, cy: block(jb, cy, True), 0)
    block(NBLK - 1, 0, False)
    for k in range(K):
        j = (NBLK - 1) * K + k
        _scat(rows, acc, dstv, ssem, j, k).wait()
        dscat(j, k).wait()
    plsc.subcore_barrier()
    pltpu.sync_copy(acc.at[pl.ds(s * RPT, RPT)],
                    parts_hbm.at[c, pl.ds(s * RPT, RPT)])
    pltpu.sync_copy(dacc.at[pl.ds(s * RPT, RPT)],
                    degp_hbm.at[c, pl.ds(s * RPT, RPT)])


_SC_MESH = dict(core_axis_name="c", subcore_axis_name="s")


def _make_sc_seg_sum(dout, K=2):
    import functools
    return pl.kernel(
        functools.partial(_sc_body, K),
        out_type=jax.ShapeDtypeStruct((NC, NP, dout), jnp.float32),
        mesh=plsc.VectorSubcoreMesh(**_SC_MESH),
        scratch_types=[
            pltpu.VMEM((NCHUNK, CH), jnp.int32),
            pltpu.VMEM((NCHUNK, CH), jnp.int32),
            pltpu.VMEM((K, CH, dout), jnp.float32),
            pltpu.VMEM_SHARED((NP, dout), jnp.float32),
        ] + [pltpu.SemaphoreType.DMA] * (2 * K),
        compiler_params=pltpu.CompilerParams(use_tc_tiling_on_sc=False),
    )


def _make_sc0(K=2):
    import functools
    return pl.kernel(
        functools.partial(_sc0_body, K),
        out_type=(jax.ShapeDtypeStruct((NC, NP, H), jnp.float32),
                  jax.ShapeDtypeStruct((NP,), jnp.float32)),
        mesh=plsc.VectorSubcoreMesh(**_SC_MESH),
        scratch_types=[
            pltpu.VMEM((NCH0, CH0), jnp.int32),
            pltpu.VMEM((NCH0, CH0), jnp.int32),
            pltpu.VMEM((K, CH0, H), jnp.float32),
            pltpu.VMEM((CH0,), jnp.float32),
            pltpu.VMEM_SHARED((NP, H), jnp.float32),
            pltpu.VMEM_SHARED((NP,), jnp.float32),
        ] + [pltpu.SemaphoreType.DMA] * (3 * K),
        compiler_params=pltpu.CompilerParams(use_tc_tiling_on_sc=False),
    )


def _make_sc_seg_sum_deg(dout, K=4):
    import functools
    return pl.kernel(
        functools.partial(_sc_body_deg, K),
        out_type=(jax.ShapeDtypeStruct((NC, NP, dout), jnp.float32),
                  jax.ShapeDtypeStruct((NC, NP), jnp.float32)),
        mesh=plsc.VectorSubcoreMesh(**_SC_MESH),
        scratch_types=[
            pltpu.VMEM((NCHUNK, CH), jnp.int32),
            pltpu.VMEM((NCHUNK, CH), jnp.int32),
            pltpu.VMEM((K, CH, dout), jnp.float32),
            pltpu.VMEM((CH,), jnp.float32),
            pltpu.VMEM_SHARED((NP, dout), jnp.float32),
            pltpu.VMEM_SHARED((NP,), jnp.float32),
        ] + [pltpu.SemaphoreType.DMA] * (3 * K),
        compiler_params=pltpu.CompilerParams(use_tc_tiling_on_sc=False),
    )


def _inv_deg(deg_ref):
    return (1.0 / jnp.maximum(deg_ref[...], 1.0))[:, None]


def _layer_body(h_ref, parts_ref, degp_ref, wst_ref, wnt_ref, b_ref, out_ref):
    mean = (parts_ref[0] + parts_ref[1]) * _inv_deg(degp_ref)
    h = h_ref[...]
    z = (jnp.dot(h, wst_ref[...], preferred_element_type=jnp.float32)
         + jnp.dot(mean, wnt_ref[...], preferred_element_type=jnp.float32)
         + b_ref[...])
    out_ref[...] = jnp.maximum(z, 0.0)


def _layer0_body(h_ref, parts_ref, degp_ref, wst_ref, wnta_ref,
                 wntb_ref, b_ref, out_ref):
    inv = _inv_deg(degp_ref)
    mean_a = parts_ref[0] * inv
    mean_b = parts_ref[1] * inv
    z = (jnp.dot(h_ref[...], wst_ref[...], preferred_element_type=jnp.float32)
         + jnp.dot(mean_a, wnta_ref[...], preferred_element_type=jnp.float32)
         + jnp.dot(mean_b, wntb_ref[...], preferred_element_type=jnp.float32)
         + b_ref[...])
    out_ref[...] = jnp.maximum(z, 0.0)


def _last_body(h_ref, parts_ref, degp_ref, wst_ref, wnt_ref, b_ref, out_ref):
    mean = (parts_ref[0] + parts_ref[1]) * _inv_deg(degp_ref)
    h = h_ref[...]
    z = (jnp.dot(h, wst_ref[...], preferred_element_type=jnp.float32)
         + jnp.dot(mean, wnt_ref[...], preferred_element_type=jnp.float32)
         + b_ref[...])
    out_ref[...] = z[:, 0]


def _row_spec(din):
    return pl.BlockSpec((BN, din), lambda i: (i, 0))


def _w_spec(din, dout):
    return pl.BlockSpec((din, dout), lambda i: (0, 0))


_V1 = pl.BlockSpec((BN,), lambda i: (i,))


def _tc_layer0(x, parts, deg, wst, wnta, wntb, b):
    return pl.pallas_call(
        _layer0_body,
        grid=(GRID,),
        in_specs=[_row_spec(D),
                  pl.BlockSpec((NC, BN, H), lambda i: (0, i, 0)),
                  _V1,
                  _w_spec(D, H), _w_spec(H, H), _w_spec(H, H),
                  pl.BlockSpec((1, H), lambda i: (0, 0))],
        out_specs=_row_spec(H),
        out_shape=jax.ShapeDtypeStruct((N, H), jnp.float32),
    )(x, parts, deg, wst, wnta, wntb, b)


def _tc_layer(h, parts, degp, wst, wnt, b, last=False):
    din, dout = wst.shape
    out_spec = _V1 if last else _row_spec(dout)
    out_shape = (jax.ShapeDtypeStruct((N,), jnp.float32) if last
                 else jax.ShapeDtypeStruct((N, dout), jnp.float32))
    return pl.pallas_call(
        _last_body if last else _layer_body,
        grid=(GRID,),
        in_specs=[_row_spec(din),
                  pl.BlockSpec((NC, BN, din), lambda i: (0, i, 0)),
                  _V1,
                  _w_spec(din, dout), _w_spec(din, dout),
                  pl.BlockSpec((1, dout), lambda i: (0, 0))],
        out_specs=out_spec,
        out_shape=out_shape,
    )(h, parts, degp, wst, wnt, b)


@jax.jit
def kernel(x, edge_index, Ws0, Wn0, b0, Ws1, Wn1, b1, Ws2, Wn2, b2,
           Ws3, Wn3, b3):
    src = edge_index[0].reshape(NW, NCHUNK, CH)
    dst = edge_index[1].reshape(NW, NCHUNK, CH)
    src0 = edge_index[0].reshape(NS, NCH0, CH0)
    dst0 = edge_index[1].reshape(NS, NCH0, CH0)


    z64 = jnp.zeros((RPT, H), jnp.float32)
    z1 = jnp.zeros((RPT,), jnp.float32)
    ones1 = jnp.ones((CH0,), jnp.float32)

    wst3 = jnp.pad(Ws3.T, ((0, 0), (0, 7)))
    wnt3 = jnp.pad(Wn3.T, ((0, 0), (0, 7)))
    b3p = jnp.pad(b3.reshape(1, 1), ((0, 0), (0, 7)))

    sc0 = _make_sc0(K=2)
    sc64 = _make_sc_seg_sum(H, K=2)

    wnt0 = Wn0.T
    parts0, deg = sc0(x[:, :H], x[:, H:], src0, dst0, z64, z1, ones1)
    h1 = _tc_layer0(x, parts0, deg, Ws0.T, wnt0[:H], wnt0[H:],
                    b0.reshape(1, H))
    parts1 = sc64(h1, src, dst, z64)
    h2 = _tc_layer(h1, parts1, deg, Ws1.T, Wn1.T, b1.reshape(1, H))
    parts2 = sc64(h2, src, dst, z64)
    h3 = _tc_layer(h2, parts2, deg, Ws2.T, Wn2.T, b2.reshape(1, H))
    parts3 = sc64(h3, src, dst, z64)
    return _tc_layer(h3, parts3, deg, wst3, wnt3, b3p, last=True)


# speedup vs baseline: 1.1601x; 1.1601x over previous
import jax
import jax.numpy as jnp
from jax import lax
from jax.experimental import pallas as pl
from jax.experimental.pallas import tpu as pltpu
from jax.experimental.pallas import tpu_sc as plsc

N = 10000
E = 320000
D = 128
H = 64

NC = 2
NS = 16
NW = NC * NS
EPW = E // NW
CH = 250
NCHUNK = EPW // CH
CH0 = 125
NCH0 = (E // NS) // CH0
NP = 10240
RPT = NP // NS

K = 8

BN = 2048
GRID = NP // BN


def _gather(hn_hbm, srcv, rows, gsem, j, k):
    return pltpu.make_async_copy(hn_hbm.at[srcv.at[j]], rows.at[k], gsem[k])


def _scat(rows, acc, dstv, ssem, j, k):
    return pltpu.make_async_copy(rows.at[k], acc.at[dstv.at[j]], ssem[k])


def _sc_body(K, hn_hbm, src_hbm, dst_hbm, z_hbm,
             parts_hbm,
             srcv, dstv, rows, acc, *sems):
    NBLK = NCHUNK // K
    gsem, ssem = sems[:K], sems[K:2 * K]
    c = lax.axis_index("c")
    s = lax.axis_index("s")
    wid = c * NS + s
    pltpu.sync_copy(src_hbm.at[wid], srcv)
    pltpu.sync_copy(dst_hbm.at[wid], dstv)
    pltpu.sync_copy(z_hbm, acc.at[pl.ds(s * RPT, RPT)])
    for k in range(K):
        _gather(hn_hbm, srcv, rows, gsem, k, k).start()
    plsc.subcore_barrier()

    def block(jb, carry, issue_next):
        base = jb * K
        for k in range(K):
            j = base + k
            _gather(hn_hbm, srcv, rows, gsem, j, k).wait()
            _scat(rows, acc, dstv, ssem, j, k).start(add=True)
        if issue_next:
            for k in range(K):
                j = base + k
                _scat(rows, acc, dstv, ssem, j, k).wait()
                _gather(hn_hbm, srcv, rows, gsem, j + K, k).start()
        return carry

    lax.fori_loop(0, NBLK - 1, lambda jb, cy: block(jb, cy, True), 0)
    block(NBLK - 1, 0, False)
    for k in range(K):
        _scat(rows, acc, dstv, ssem, (NBLK - 1) * K + k, k).wait()
    plsc.subcore_barrier()
    pltpu.sync_copy(acc.at[pl.ds(s * RPT, RPT)],
                    parts_hbm.at[c, pl.ds(s * RPT, RPT)])


def _sc0_body(K, xlo_hbm, xhi_hbm, src_hbm, dst_hbm, z_hbm, z1_hbm,
              ones_hbm,
              parts_hbm, deg_hbm,
              srcv, dstv, rows, onesv, acc, dacc, *sems):
    NBLK = NCH0 // K
    gsem, ssem, dsem = sems[:K], sems[K:2 * K], sems[2 * K:3 * K]
    c = lax.axis_index("c")
    s = lax.axis_index("s")
    pltpu.sync_copy(src_hbm.at[s], srcv)
    pltpu.sync_copy(dst_hbm.at[s], dstv)
    pltpu.sync_copy(ones_hbm, onesv)
    pltpu.sync_copy(z_hbm, acc.at[pl.ds(s * RPT, RPT)])
    pltpu.sync_copy(z1_hbm, dacc.at[pl.ds(s * RPT, RPT)])

    def dscat(j, k):
        return pltpu.make_async_copy(onesv, dacc.at[dstv.at[j]], dsem[k])

    def pipe(xc, with_deg):
        for k in range(K):
            _gather(xc, srcv, rows, gsem, k, k).start()
        plsc.subcore_barrier()

        def block(jb, carry, issue_next):
            base = jb * K
            for k in range(K):
                j = base + k
                _gather(xc, srcv, rows, gsem, j, k).wait()
                _scat(rows, acc, dstv, ssem, j, k).start(add=True)
                if with_deg:
                    dscat(j, k).start(add=True)
            if issue_next:
                for k in range(K):
                    j = base + k
                    _scat(rows, acc, dstv, ssem, j, k).wait()
                    if with_deg:
                        dscat(j, k).wait()
                    _gather(xc, srcv, rows, gsem, j + K, k).start()
            return carry

        lax.fori_loop(0, NBLK - 1, lambda jb, cy: block(jb, cy, True), 0)
        block(NBLK - 1, 0, False)
        for k in range(K):
            j = (NBLK - 1) * K + k
            _scat(rows, acc, dstv, ssem, j, k).wait()
            if with_deg:
                dscat(j, k).wait()
        plsc.subcore_barrier()

    @pl.when(c == 0)
    def _():
        pipe(xlo_hbm, True)

    @pl.when(c == 1)
    def _():
        pipe(xhi_hbm, False)
    pltpu.sync_copy(acc.at[pl.ds(s * RPT, RPT)],
                    parts_hbm.at[c, pl.ds(s * RPT, RPT)])

    @pl.when(c == 0)
    def _():
        pltpu.sync_copy(dacc.at[pl.ds(s * RPT, RPT)],
                        deg_hbm.at[pl.ds(s * RPT, RPT)])


def _sc_body_deg(K, hn_hbm, src_hbm, dst_hbm, z_hbm, z1_hbm, ones_hbm,
                 parts_hbm, degp_hbm,
                 srcv, dstv, rows, onesv, acc, dacc, *sems):
    NBLK = NCHUNK // K
    gsem, ssem, dsem = sems[:K], sems[K:2 * K], sems[2 * K:3 * K]
    c = lax.axis_index("c")
    s = lax.axis_index("s")
    wid = c * NS + s
    pltpu.sync_copy(src_hbm.at[wid], srcv)
    pltpu.sync_copy(dst_hbm.at[wid], dstv)
    pltpu.sync_copy(ones_hbm, onesv)
    pltpu.sync_copy(z_hbm, acc.at[pl.ds(s * RPT, RPT)])
    pltpu.sync_copy(z1_hbm, dacc.at[pl.ds(s * RPT, RPT)])
    for k in range(K):
        _gather(hn_hbm, srcv, rows, gsem, k, k).start()
    plsc.subcore_barrier()

    def dscat(j, k):
        return pltpu.make_async_copy(onesv, dacc.at[dstv.at[j]], dsem[k])

    def block(jb, carry, issue_next):
        base = jb * K
        for k in range(K):
            j = base + k
            _gather(hn_hbm, srcv, rows, gsem, j, k).wait()
            _scat(rows, acc, dstv, ssem, j, k).start(add=True)
            dscat(j, k).start(add=True)
        if issue_next:
            for k in range(K):
                j = base + k
                _scat(rows, acc, dstv, ssem, j, k).wait()
                dscat(j, k).wait()
                _gather(hn_hbm, srcv, rows, gsem, j + K, k).start()
        return carry

    lax.fori_loop(0, NBLK - 1, lambda jb, cy: block(jb, cy, True), 0)
    block(NBLK - 1, 0, False)
    for k in range(K):
        j = (NBLK - 1) * K + k
        _scat(rows, acc, dstv, ssem, j, k).wait()
        dscat(j, k).wait()
    plsc.subcore_barrier()
    pltpu.sync_copy(acc.at[pl.ds(s * RPT, RPT)],
                    parts_hbm.at[c, pl.ds(s * RPT, RPT)])
    pltpu.sync_copy(dacc.at[pl.ds(s * RPT, RPT)],
                    degp_hbm.at[c, pl.ds(s * RPT, RPT)])


_SC_MESH = dict(core_axis_name="c", subcore_axis_name="s")


def _make_sc_seg_sum(dout, K=2):
    import functools
    return pl.kernel(
        functools.partial(_sc_body, K),
        out_type=jax.ShapeDtypeStruct((NC, NP, dout), jnp.float32),
        mesh=plsc.VectorSubcoreMesh(**_SC_MESH),
        scratch_types=[
            pltpu.VMEM((NCHUNK, CH), jnp.int32),
            pltpu.VMEM((NCHUNK, CH), jnp.int32),
            pltpu.VMEM((K, CH, dout), jnp.float32),
            pltpu.VMEM_SHARED((NP, dout), jnp.float32),
        ] + [pltpu.SemaphoreType.DMA] * (2 * K),
        compiler_params=pltpu.CompilerParams(use_tc_tiling_on_sc=False),
    )


def _make_sc0(K=2):
    import functools
    return pl.kernel(
        functools.partial(_sc0_body, K),
        out_type=(jax.ShapeDtypeStruct((NC, NP, H), jnp.float32),
                  jax.ShapeDtypeStruct((NP,), jnp.float32)),
        mesh=plsc.VectorSubcoreMesh(**_SC_MESH),
        scratch_types=[
            pltpu.VMEM((NCH0, CH0), jnp.int32),
            pltpu.VMEM((NCH0, CH0), jnp.int32),
            pltpu.VMEM((K, CH0, H), jnp.float32),
            pltpu.VMEM((CH0,), jnp.float32),
            pltpu.VMEM_SHARED((NP, H), jnp.float32),
            pltpu.VMEM_SHARED((NP,), jnp.float32),
        ] + [pltpu.SemaphoreType.DMA] * (3 * K),
        compiler_params=pltpu.CompilerParams(use_tc_tiling_on_sc=False),
    )


def _make_sc_seg_sum_deg(dout, K=4):
    import functools
    return pl.kernel(
        functools.partial(_sc_body_deg, K),
        out_type=(jax.ShapeDtypeStruct((NC, NP, dout), jnp.float32),
                  jax.ShapeDtypeStruct((NC, NP), jnp.float32)),
        mesh=plsc.VectorSubcoreMesh(**_SC_MESH),
        scratch_types=[
            pltpu.VMEM((NCHUNK, CH), jnp.int32),
            pltpu.VMEM((NCHUNK, CH), jnp.int32),
            pltpu.VMEM((K, CH, dout), jnp.float32),
            pltpu.VMEM((CH,), jnp.float32),
            pltpu.VMEM_SHARED((NP, dout), jnp.float32),
            pltpu.VMEM_SHARED((NP,), jnp.float32),
        ] + [pltpu.SemaphoreType.DMA] * (3 * K),
        compiler_params=pltpu.CompilerParams(use_tc_tiling_on_sc=False),
    )


def _inv_deg(deg_ref):
    return (1.0 / jnp.maximum(deg_ref[...], 1.0))[:, None]


def _layer_body(h_ref, parts_ref, degp_ref, wst_ref, wnt_ref, b_ref, out_ref):
    mean = (parts_ref[0] + parts_ref[1]) * _inv_deg(degp_ref)
    h = h_ref[...]
    z = (jnp.dot(h, wst_ref[...], preferred_element_type=jnp.float32)
         + jnp.dot(mean, wnt_ref[...], preferred_element_type=jnp.float32)
         + b_ref[...])
    out_ref[...] = jnp.maximum(z, 0.0)


def _layer0_body(h_ref, parts_ref, degp_ref, wst_ref, wnta_ref,
                 wntb_ref, b_ref, out_ref):
    inv = _inv_deg(degp_ref)
    mean_a = parts_ref[0] * inv
    mean_b = parts_ref[1] * inv
    z = (jnp.dot(h_ref[...], wst_ref[...], preferred_element_type=jnp.float32)
         + jnp.dot(mean_a, wnta_ref[...], preferred_element_type=jnp.float32)
         + jnp.dot(mean_b, wntb_ref[...], preferred_element_type=jnp.float32)
         + b_ref[...])
    out_ref[...] = jnp.maximum(z, 0.0)


def _last_body(h_ref, parts_ref, degp_ref, wst_ref, wnt_ref, b_ref, out_ref):
    mean = (parts_ref[0] + parts_ref[1]) * _inv_deg(degp_ref)
    h = h_ref[...]
    z = (jnp.dot(h, wst_ref[...], preferred_element_type=jnp.float32)
         + jnp.dot(mean, wnt_ref[...], preferred_element_type=jnp.float32)
         + b_ref[...])
    out_ref[...] = z[:, 0]


def _row_spec(din):
    return pl.BlockSpec((BN, din), lambda i: (i, 0))


def _w_spec(din, dout):
    return pl.BlockSpec((din, dout), lambda i: (0, 0))


_V1 = pl.BlockSpec((BN,), lambda i: (i,))


def _tc_layer0(x, parts, deg, wst, wnta, wntb, b):
    return pl.pallas_call(
        _layer0_body,
        grid=(GRID,),
        in_specs=[_row_spec(D),
                  pl.BlockSpec((NC, BN, H), lambda i: (0, i, 0)),
                  _V1,
                  _w_spec(D, H), _w_spec(H, H), _w_spec(H, H),
                  pl.BlockSpec((1, H), lambda i: (0, 0))],
        out_specs=_row_spec(H),
        out_shape=jax.ShapeDtypeStruct((N, H), jnp.float32),
    )(x, parts, deg, wst, wnta, wntb, b)


def _tc_layer(h, parts, degp, wst, wnt, b, last=False):
    din, dout = wst.shape
    out_spec = _V1 if last else _row_spec(dout)
    out_shape = (jax.ShapeDtypeStruct((N,), jnp.float32) if last
                 else jax.ShapeDtypeStruct((N, dout), jnp.float32))
    return pl.pallas_call(
        _last_body if last else _layer_body,
        grid=(GRID,),
        in_specs=[_row_spec(din),
                  pl.BlockSpec((NC, BN, din), lambda i: (0, i, 0)),
                  _V1,
                  _w_spec(din, dout), _w_spec(din, dout),
                  pl.BlockSpec((1, dout), lambda i: (0, 0))],
        out_specs=out_spec,
        out_shape=out_shape,
    )(h, parts, degp, wst, wnt, b)


@jax.jit
def kernel(x, edge_index, Ws0, Wn0, b0, Ws1, Wn1, b1, Ws2, Wn2, b2,
           Ws3, Wn3, b3):
    src = edge_index[0].reshape(NW, NCHUNK, CH)
    dst = edge_index[1].reshape(NW, NCHUNK, CH)
    src0 = edge_index[0].reshape(NS, NCH0, CH0)
    dst0 = edge_index[1].reshape(NS, NCH0, CH0)


    z64 = jnp.zeros((RPT, H), jnp.float32)
    z1 = jnp.zeros((RPT,), jnp.float32)
    ones1 = jnp.ones((CH0,), jnp.float32)

    wst3 = jnp.pad(Ws3.T, ((0, 0), (0, 7)))
    wnt3 = jnp.pad(Wn3.T, ((0, 0), (0, 7)))
    b3p = jnp.pad(b3.reshape(1, 1), ((0, 0), (0, 7)))

    sc0 = _make_sc0(K=4)
    sc64 = _make_sc_seg_sum(H, K=4)

    wnt0 = Wn0.T
    parts0, deg = sc0(x[:, :H], x[:, H:], src0, dst0, z64, z1, ones1)
    h1 = _tc_layer0(x, parts0, deg, Ws0.T, wnt0[:H], wnt0[H:],
                    b0.reshape(1, H))
    parts1 = sc64(h1, src, dst, z64)
    h2 = _tc_layer(h1, parts1, deg, Ws1.T, Wn1.T, b1.reshape(1, H))
    parts2 = sc64(h2, src, dst, z64)
    h3 = _tc_layer(h2, parts2, deg, Ws2.T, Wn2.T, b2.reshape(1, H))
    parts3 = sc64(h3, src, dst, z64)
    return _tc_layer(h3, parts3, deg, wst3, wnt3, b3p, last=True)


# speedup vs baseline: 1.1818x; 1.0187x over previous
import functools

import jax
import jax.numpy as jnp
from jax import lax
from jax.experimental import pallas as pl
from jax.experimental.pallas import tpu as pltpu
from jax.experimental.pallas import tpu_sc as plsc

N = 10000
E = 320000
D = 128
H = 64

NC = 2
NS = 16
NW = NC * NS
EPW = E // NW
CH = 125
NCHUNK = EPW // CH
NP = 10240
RPT = NP // NS

K = 8

BN = 2048
GRID = NP // BN


def _gather(hn_hbm, srcv, rows, gsem, j, k):
    return pltpu.make_async_copy(hn_hbm.at[srcv.at[j]], rows.at[k], gsem[k])


def _scat(rows, acc, dstv, ssem, j, k):
    return pltpu.make_async_copy(rows.at[k], acc.at[dstv.at[j]], ssem[k])


def _sc_body(K, hn_hbm, src_hbm, dst_hbm, z_hbm,
             parts_hbm,
             srcv, dstv, rows, acc, *sems):
    NBLK = NCHUNK // K
    gsem, ssem = sems[:K], sems[K:2 * K]
    c = lax.axis_index("c")
    s = lax.axis_index("s")
    wid = c * NS + s
    pltpu.sync_copy(src_hbm.at[wid], srcv)
    pltpu.sync_copy(dst_hbm.at[wid], dstv)
    pltpu.sync_copy(z_hbm, acc.at[pl.ds(s * RPT, RPT)])
    for k in range(K):
        _gather(hn_hbm, srcv, rows, gsem, k, k).start()
    plsc.subcore_barrier()

    def block(jb, carry, issue_next):
        base = jb * K
        for k in range(K):
            j = base + k
            _gather(hn_hbm, srcv, rows, gsem, j, k).wait()
            _scat(rows, acc, dstv, ssem, j, k).start(add=True)
        if issue_next:
            for k in range(K):
                j = base + k
                _scat(rows, acc, dstv, ssem, j, k).wait()
                _gather(hn_hbm, srcv, rows, gsem, j + K, k).start()
        return carry

    lax.fori_loop(0, NBLK - 1, lambda jb, cy: block(jb, cy, True), 0)
    block(NBLK - 1, 0, False)
    for k in range(K):
        _scat(rows, acc, dstv, ssem, (NBLK - 1) * K + k, k).wait()
    plsc.subcore_barrier()
    pltpu.sync_copy(acc.at[pl.ds(s * RPT, RPT)],
                    parts_hbm.at[c, pl.ds(s * RPT, RPT)])


def _sc0_body(K, xlo_hbm, xhi_hbm, src_hbm, dst_hbm, z_hbm, z1_hbm,
              ones_hbm,
              parts_hbm, deg_hbm,
              srcv, dstv, rows, onesv, acc, dacc, *sems):
    NB = (E // NS) // CH
    NBLK = NB // K
    gsem, ssem, dsem = sems[:K], sems[K:2 * K], sems[2 * K:3 * K]
    c = lax.axis_index("c")
    s = lax.axis_index("s")
    pltpu.sync_copy(src_hbm.at[s], srcv)
    pltpu.sync_copy(dst_hbm.at[s], dstv)
    pltpu.sync_copy(ones_hbm, onesv)
    pltpu.sync_copy(z_hbm, acc.at[pl.ds(s * RPT, RPT)])
    pltpu.sync_copy(z1_hbm, dacc.at[pl.ds(s * RPT, RPT)])

    def dscat(j, k):
        return pltpu.make_async_copy(onesv, dacc.at[dstv.at[j]], dsem[k])

    def pipe(xc, with_deg):
        for k in range(K):
            _gather(xc, srcv, rows, gsem, k, k).start()
        plsc.subcore_barrier()

        def block(jb, carry, issue_next):
            base = jb * K
            for k in range(K):
                j = base + k
                _gather(xc, srcv, rows, gsem, j, k).wait()
                _scat(rows, acc, dstv, ssem, j, k).start(add=True)
                if with_deg:
                    dscat(j, k).start(add=True)
            if issue_next:
                for k in range(K):
                    j = base + k
                    _scat(rows, acc, dstv, ssem, j, k).wait()
                    if with_deg:
                        dscat(j, k).wait()
                    _gather(xc, srcv, rows, gsem, j + K, k).start()
            return carry

        lax.fori_loop(0, NBLK - 1, lambda jb, cy: block(jb, cy, True), 0)
        block(NBLK - 1, 0, False)
        for k in range(K):
            j = (NBLK - 1) * K + k
            _scat(rows, acc, dstv, ssem, j, k).wait()
            if with_deg:
                dscat(j, k).wait()
        plsc.subcore_barrier()

    @pl.when(c == 0)
    def _():
        pipe(xlo_hbm, True)

    @pl.when(c == 1)
    def _():
        pipe(xhi_hbm, False)
    pltpu.sync_copy(acc.at[pl.ds(s * RPT, RPT)],
                    parts_hbm.at[c, pl.ds(s * RPT, RPT)])

    @pl.when(c == 0)
    def _():
        pltpu.sync_copy(dacc.at[pl.ds(s * RPT, RPT)],
                        deg_hbm.at[pl.ds(s * RPT, RPT)])


_SC_MESH = dict(core_axis_name="c", subcore_axis_name="s")


def _make_sc_seg_sum(dout, K=K):
    return pl.kernel(
        functools.partial(_sc_body, K),
        out_type=jax.ShapeDtypeStruct((NC, NP, dout), jnp.float32),
        mesh=plsc.VectorSubcoreMesh(**_SC_MESH),
        scratch_types=[
            pltpu.VMEM((NCHUNK, CH), jnp.int32),
            pltpu.VMEM((NCHUNK, CH), jnp.int32),
            pltpu.VMEM((K, CH, dout), jnp.float32),
            pltpu.VMEM_SHARED((NP, dout), jnp.float32),
        ] + [pltpu.SemaphoreType.DMA] * (2 * K),
        compiler_params=pltpu.CompilerParams(use_tc_tiling_on_sc=False),
    )


def _make_sc0(K=8):
    NB = (E // NS) // CH
    return pl.kernel(
        functools.partial(_sc0_body, K),
        out_type=(jax.ShapeDtypeStruct((NC, NP, H), jnp.float32),
                  jax.ShapeDtypeStruct((NP,), jnp.float32)),
        mesh=plsc.VectorSubcoreMesh(**_SC_MESH),
        scratch_types=[
            pltpu.VMEM((NB, CH), jnp.int32),
            pltpu.VMEM((NB, CH), jnp.int32),
            pltpu.VMEM((K, CH, H), jnp.float32),
            pltpu.VMEM((CH,), jnp.float32),
            pltpu.VMEM_SHARED((NP, H), jnp.float32),
            pltpu.VMEM_SHARED((NP,), jnp.float32),
        ] + [pltpu.SemaphoreType.DMA] * (3 * K),
        compiler_params=pltpu.CompilerParams(use_tc_tiling_on_sc=False),
    )


def _inv_deg(deg_ref):
    return (1.0 / jnp.maximum(deg_ref[...], 1.0))[:, None]


def _layer_body(h_ref, parts_ref, degp_ref, wst_ref, wnt_ref, b_ref, out_ref):
    mean = (parts_ref[0] + parts_ref[1]) * _inv_deg(degp_ref)
    h = h_ref[...]
    z = (jnp.dot(h, wst_ref[...], preferred_element_type=jnp.float32)
         + jnp.dot(mean, wnt_ref[...], preferred_element_type=jnp.float32)
         + b_ref[...])
    out_ref[...] = jnp.maximum(z, 0.0)


def _layer0_body(h_ref, parts_ref, degp_ref, wst_ref, wnta_ref,
                 wntb_ref, b_ref, out_ref):
    inv = _inv_deg(degp_ref)
    mean_a = parts_ref[0] * inv
    mean_b = parts_ref[1] * inv
    z = (jnp.dot(h_ref[...], wst_ref[...], preferred_element_type=jnp.float32)
         + jnp.dot(mean_a, wnta_ref[...], preferred_element_type=jnp.float32)
         + jnp.dot(mean_b, wntb_ref[...], preferred_element_type=jnp.float32)
         + b_ref[...])
    out_ref[...] = jnp.maximum(z, 0.0)


def _last_body(h_ref, parts_ref, degp_ref, wst_ref, wnt_ref, b_ref, out_ref):
    mean = (parts_ref[0] + parts_ref[1]) * _inv_deg(degp_ref)
    h = h_ref[...]
    z = (jnp.dot(h, wst_ref[...], preferred_element_type=jnp.float32)
         + jnp.dot(mean, wnt_ref[...], preferred_element_type=jnp.float32)
         + b_ref[...])
    out_ref[...] = z[:, 0]


def _row_spec(din):
    return pl.BlockSpec((BN, din), lambda i: (i, 0))


def _w_spec(din, dout):
    return pl.BlockSpec((din, dout), lambda i: (0, 0))


_V1 = pl.BlockSpec((BN,), lambda i: (i,))


def _tc_layer0(x, parts, deg, wst, wnta, wntb, b):
    return pl.pallas_call(
        _layer0_body,
        grid=(GRID,),
        in_specs=[_row_spec(D),
                  pl.BlockSpec((NC, BN, H), lambda i: (0, i, 0)),
                  _V1,
                  _w_spec(D, H), _w_spec(H, H), _w_spec(H, H),
                  pl.BlockSpec((1, H), lambda i: (0, 0))],
        out_specs=_row_spec(H),
        out_shape=jax.ShapeDtypeStruct((N, H), jnp.float32),
    )(x, parts, deg, wst, wnta, wntb, b)


def _tc_layer(h, parts, degp, wst, wnt, b, last=False):
    din, dout = wst.shape
    out_spec = _V1 if last else _row_spec(dout)
    out_shape = (jax.ShapeDtypeStruct((N,), jnp.float32) if last
                 else jax.ShapeDtypeStruct((N, dout), jnp.float32))
    return pl.pallas_call(
        _last_body if last else _layer_body,
        grid=(GRID,),
        in_specs=[_row_spec(din),
                  pl.BlockSpec((NC, BN, din), lambda i: (0, i, 0)),
                  _V1,
                  _w_spec(din, dout), _w_spec(din, dout),
                  pl.BlockSpec((1, dout), lambda i: (0, 0))],
        out_specs=out_spec,
        out_shape=out_shape,
    )(h, parts, degp, wst, wnt, b)


@jax.jit
def kernel(x, edge_index, Ws0, Wn0, b0, Ws1, Wn1, b1, Ws2, Wn2, b2,
           Ws3, Wn3, b3):
    src = edge_index[0].reshape(NW, NCHUNK, CH)
    dst = edge_index[1].reshape(NW, NCHUNK, CH)
    NB0 = (E // NS) // CH
    src0 = edge_index[0].reshape(NS, NB0, CH)
    dst0 = edge_index[1].reshape(NS, NB0, CH)


    z64 = jnp.zeros((RPT, H), jnp.float32)
    z1 = jnp.zeros((RPT,), jnp.float32)
    ones1 = jnp.ones((CH,), jnp.float32)

    wst3 = jnp.pad(Ws3.T, ((0, 0), (0, 7)))
    wnt3 = jnp.pad(Wn3.T, ((0, 0), (0, 7)))
    b3p = jnp.pad(b3.reshape(1, 1), ((0, 0), (0, 7)))

    sc0 = _make_sc0(K=5)
    sc64 = _make_sc_seg_sum(H, K=8)

    wnt0 = Wn0.T
    parts0, deg = sc0(x[:, :H], x[:, H:], src0, dst0, z64, z1, ones1)
    h1 = _tc_layer0(x, parts0, deg, Ws0.T, wnt0[:H], wnt0[H:],
                    b0.reshape(1, H))
    parts1 = sc64(h1, src, dst, z64)
    h2 = _tc_layer(h1, parts1, deg, Ws1.T, Wn1.T, b1.reshape(1, H))
    parts2 = sc64(h2, src, dst, z64)
    h3 = _tc_layer(h2, parts2, deg, Ws2.T, Wn2.T, b2.reshape(1, H))
    parts3 = sc64(h3, src, dst, z64)
    return _tc_layer(h3, parts3, deg, wst3, wnt3, b3p, last=True)
